# baseline (speedup 1.0000x reference)
import functools

import jax
import jax.numpy as jnp
from jax.experimental import pallas as pl
from jax.experimental.pallas import tpu as pltpu

_BBOX_STD = (0.1, 0.1, 0.2, 0.2)
_MAX_OUT = 100
_NMS_THR = 0.3
_CONF = 0.15
_NEG = -1e30

_ROWS = 160
_LANES = 128
_P = _ROWS * _LANES


def _nms_body(boxes_ref, deltas_ref, probs_ref, meta_ref, out_ref,
              y1_s, x1_s, y2_s, x2_s, ar_s, s_s):
    h = meta_ref[0, 4]
    w = meta_ref[0, 5]
    wy1 = (meta_ref[0, 7] - 0.0) / (h - 1.0)
    wx1 = (meta_ref[0, 8] - 0.0) / (w - 1.0)
    wy2 = (meta_ref[0, 9] - 1.0) / (h - 1.0)
    wx2 = (meta_ref[0, 10] - 1.0) / (w - 1.0)

    ry1 = boxes_ref[0]
    rx1 = boxes_ref[1]
    ry2 = boxes_ref[2]
    rx2 = boxes_ref[3]
    dy = deltas_ref[0] * _BBOX_STD[0]
    dx = deltas_ref[1] * _BBOX_STD[1]
    dh = deltas_ref[2] * _BBOX_STD[2]
    dw = deltas_ref[3] * _BBOX_STD[3]

    height = ry2 - ry1
    width = rx2 - rx1
    cy = ry1 + 0.5 * height + dy * height
    cx = rx1 + 0.5 * width + dx * width
    height = height * jnp.exp(dh)
    width = width * jnp.exp(dw)
    y1 = cy - 0.5 * height
    x1 = cx - 0.5 * width
    y2 = y1 + height
    x2 = x1 + width

    y1 = jnp.clip(y1, wy1, wy2)
    x1 = jnp.clip(x1, wx1, wx2)
    y2 = jnp.clip(y2, wy1, wy2)
    x2 = jnp.clip(x2, wx1, wx2)

    y1_s[...] = y1
    x1_s[...] = x1
    y2_s[...] = y2
    x2_s[...] = x2
    ar_s[...] = (y2 - y1) * (x2 - x1)

    probs = probs_ref[...]
    s_s[...] = jnp.where(probs >= _CONF, probs, _NEG)

    gidx = (jax.lax.broadcasted_iota(jnp.int32, (_ROWS, _LANES), 0) * _LANES
            + jax.lax.broadcasted_iota(jnp.int32, (_ROWS, _LANES), 1))
    lane = jax.lax.broadcasted_iota(jnp.int32, (1, _LANES), 1)

    rowi = jax.lax.broadcasted_iota(jnp.int32, (_ROWS, 1), 0)

    def body(k, carry):
        s = s_s[...]
        rowmax = jnp.max(s, axis=1, keepdims=True)
        best = jnp.max(rowmax)
        valid = best > _NEG / 2
        r = jnp.min(jnp.where(rowmax == best, rowi, _ROWS))
        srow = s_s[pl.ds(r, 1), :]
        c = jnp.min(jnp.where(srow == best, lane, _LANES))
        idx = r * _LANES + c

        lm = lane == c

        def pick(ref):
            row = ref[pl.ds(r, 1), :]
            return jnp.sum(jnp.where(lm, row, 0.0))

        y1b = pick(y1_s)
        x1b = pick(x1_s)
        y2b = pick(y2_s)
        x2b = pick(x2_s)
        area_b = pick(ar_s)

        y1a = y1_s[...]
        x1a = x1_s[...]
        y2a = y2_s[...]
        x2a = x2_s[...]
        areas = ar_s[...]
        yy1 = jnp.maximum(y1b, y1a)
        xx1 = jnp.maximum(x1b, x1a)
        yy2 = jnp.minimum(y2b, y2a)
        xx2 = jnp.minimum(x2b, x2a)
        inter = jnp.maximum(yy2 - yy1, 0.0) * jnp.maximum(xx2 - xx1, 0.0)
        iou = inter / (area_b + areas - inter + 1e-9)
        suppress = (iou > _NMS_THR) | (gidx == idx)
        s_s[...] = jnp.where(valid & suppress, _NEG, s)

        zf = jnp.float32(0.0)
        o0 = jnp.where(valid, y1b, zf)
        o1 = jnp.where(valid, x1b, zf)
        o2 = jnp.where(valid, y2b, zf)
        o3 = jnp.where(valid, x2b, zf)
        o4 = jnp.where(valid, best, zf)
        row = jnp.where(
            lane == 0, o0,
            jnp.where(lane == 1, o1,
                      jnp.where(lane == 2, o2,
                                jnp.where(lane == 3, o3,
                                          jnp.where(lane == 4, o4, zf)))))
        out_ref[pl.ds(k, 1), :] = row
        return carry

    jax.lax.fori_loop(0, _MAX_OUT, body, 0)


@functools.partial(jax.jit, static_argnames=())
def kernel(rois, bbox_scores, macacnn_bbox, image_meta):
    B, N = rois.shape[0], rois.shape[1]
    pad = _P - N

    def prep(x, fill):
        xt = jnp.transpose(x, (1, 0))
        xt = jnp.pad(xt, ((0, 0), (0, pad)), constant_values=fill)
        return xt.reshape(x.shape[1], _ROWS, _LANES)

    boxes_in = prep(rois[0], 0.0)
    deltas_in = prep(macacnn_bbox[0], 0.0)
    probs_in = prep(bbox_scores[0], -1.0)[0]

    out = pl.pallas_call(
        _nms_body,
        out_shape=jax.ShapeDtypeStruct((_MAX_OUT, _LANES), jnp.float32),
        in_specs=[
            pl.BlockSpec(memory_space=pltpu.VMEM),
            pl.BlockSpec(memory_space=pltpu.VMEM),
            pl.BlockSpec(memory_space=pltpu.VMEM),
            pl.BlockSpec(memory_space=pltpu.SMEM),
        ],
        out_specs=pl.BlockSpec(memory_space=pltpu.VMEM),
        scratch_shapes=[pltpu.VMEM((_ROWS, _LANES), jnp.float32)] * 6,
    )(boxes_in, deltas_in, probs_in, image_meta)

    return out[:, :5].reshape(B, _MAX_OUT, 5)


# speedup vs baseline: 1.0838x; 1.0838x over previous
import functools

import jax
import jax.numpy as jnp
from jax.experimental import pallas as pl
from jax.experimental.pallas import tpu as pltpu

_BBOX_STD = (0.1, 0.1, 0.2, 0.2)
_MAX_OUT = 100
_NMS_THR = 0.3
_CONF = 0.15
_NEG = -1e30

_ROWS = 160
_LANES = 128
_P = _ROWS * _LANES


def _nms_body(boxes_ref, deltas_ref, probs_ref, meta_ref, out_ref,
              y1_s, x1_s, y2_s, x2_s, ar_s, s_s):
    h = meta_ref[0, 4]
    w = meta_ref[0, 5]
    wy1 = (meta_ref[0, 7] - 0.0) / (h - 1.0)
    wx1 = (meta_ref[0, 8] - 0.0) / (w - 1.0)
    wy2 = (meta_ref[0, 9] - 1.0) / (h - 1.0)
    wx2 = (meta_ref[0, 10] - 1.0) / (w - 1.0)

    ry1 = boxes_ref[0]
    rx1 = boxes_ref[1]
    ry2 = boxes_ref[2]
    rx2 = boxes_ref[3]
    dy = deltas_ref[0] * _BBOX_STD[0]
    dx = deltas_ref[1] * _BBOX_STD[1]
    dh = deltas_ref[2] * _BBOX_STD[2]
    dw = deltas_ref[3] * _BBOX_STD[3]

    height = ry2 - ry1
    width = rx2 - rx1
    cy = ry1 + 0.5 * height + dy * height
    cx = rx1 + 0.5 * width + dx * width
    height = height * jnp.exp(dh)
    width = width * jnp.exp(dw)
    y1 = cy - 0.5 * height
    x1 = cx - 0.5 * width
    y2 = y1 + height
    x2 = x1 + width

    y1 = jnp.clip(y1, wy1, wy2)
    x1 = jnp.clip(x1, wx1, wx2)
    y2 = jnp.clip(y2, wy1, wy2)
    x2 = jnp.clip(x2, wx1, wx2)

    y1_s[...] = y1
    x1_s[...] = x1
    y2_s[...] = y2
    x2_s[...] = x2
    ar_s[...] = (y2 - y1) * (x2 - x1)

    probs = probs_ref[...]
    s_s[...] = jnp.where(probs >= _CONF, probs, _NEG)

    gidx = (jax.lax.broadcasted_iota(jnp.int32, (_ROWS, _LANES), 0) * _LANES
            + jax.lax.broadcasted_iota(jnp.int32, (_ROWS, _LANES), 1))
    lane = jax.lax.broadcasted_iota(jnp.int32, (1, _LANES), 1)

    def body(k, carry):
        s = s_s[...]
        best = jnp.max(s)
        valid = best > _NEG / 2
        sel = s == best
        idx = jnp.min(jnp.where(sel, gidx, _P))
        r = idx // _LANES
        c = idx % _LANES

        lm = lane == c

        def pick(ref):
            row = ref[pl.ds(r, 1), :]
            return jnp.sum(jnp.where(lm, row, 0.0))

        y1b = pick(y1_s)
        x1b = pick(x1_s)
        y2b = pick(y2_s)
        x2b = pick(x2_s)
        area_b = pick(ar_s)

        y1a = y1_s[...]
        x1a = x1_s[...]
        y2a = y2_s[...]
        x2a = x2_s[...]
        areas = ar_s[...]
        yy1 = jnp.maximum(y1b, y1a)
        xx1 = jnp.maximum(x1b, x1a)
        yy2 = jnp.minimum(y2b, y2a)
        xx2 = jnp.minimum(x2b, x2a)
        inter = jnp.maximum(yy2 - yy1, 0.0) * jnp.maximum(xx2 - xx1, 0.0)
        iou = inter / (area_b + areas - inter + 1e-9)
        suppress = (iou > _NMS_THR) | (gidx == idx)
        s_s[...] = jnp.where(valid & suppress, _NEG, s)

        zf = jnp.float32(0.0)
        o0 = jnp.where(valid, y1b, zf)
        o1 = jnp.where(valid, x1b, zf)
        o2 = jnp.where(valid, y2b, zf)
        o3 = jnp.where(valid, x2b, zf)
        o4 = jnp.where(valid, best, zf)
        row = jnp.where(
            lane == 0, o0,
            jnp.where(lane == 1, o1,
                      jnp.where(lane == 2, o2,
                                jnp.where(lane == 3, o3,
                                          jnp.where(lane == 4, o4, zf)))))
        out_ref[pl.ds(k, 1), :] = row
        return carry

    jax.lax.fori_loop(0, _MAX_OUT, body, 0)


@functools.partial(jax.jit, static_argnames=())
def kernel(rois, bbox_scores, macacnn_bbox, image_meta):
    B, N = rois.shape[0], rois.shape[1]
    pad = _P - N

    def prep(x, fill):
        xt = jnp.transpose(x, (1, 0))
        xt = jnp.pad(xt, ((0, 0), (0, pad)), constant_values=fill)
        return xt.reshape(x.shape[1], _ROWS, _LANES)

    boxes_in = prep(rois[0], 0.0)
    deltas_in = prep(macacnn_bbox[0], 0.0)
    probs_in = prep(bbox_scores[0], -1.0)[0]

    out = pl.pallas_call(
        _nms_body,
        out_shape=jax.ShapeDtypeStruct((_MAX_OUT, _LANES), jnp.float32),
        in_specs=[
            pl.BlockSpec(memory_space=pltpu.VMEM),
            pl.BlockSpec(memory_space=pltpu.VMEM),
            pl.BlockSpec(memory_space=pltpu.VMEM),
            pl.BlockSpec(memory_space=pltpu.SMEM),
        ],
        out_specs=pl.BlockSpec(memory_space=pltpu.VMEM),
        scratch_shapes=[pltpu.VMEM((_ROWS, _LANES), jnp.float32)] * 6,
    )(boxes_in, deltas_in, probs_in, image_meta)

    return out[:, :5].reshape(B, _MAX_OUT, 5)


# speedup vs baseline: 1.0949x; 1.0103x over previous
import functools

import jax
import jax.numpy as jnp
from jax.experimental import pallas as pl
from jax.experimental.pallas import tpu as pltpu

_BBOX_STD = (0.1, 0.1, 0.2, 0.2)
_MAX_OUT = 100
_NMS_THR = 0.3
_CONF = 0.15
_NEG = -1e30

_ROWS = 160
_LANES = 128
_P = _ROWS * _LANES


def _nms_body(boxes_ref, deltas_ref, probs_ref, meta_ref, out_ref,
              y1_s, x1_s, y2_s, x2_s, ar_s, s_s):
    h = meta_ref[0, 4]
    w = meta_ref[0, 5]
    wy1 = (meta_ref[0, 7] - 0.0) / (h - 1.0)
    wx1 = (meta_ref[0, 8] - 0.0) / (w - 1.0)
    wy2 = (meta_ref[0, 9] - 1.0) / (h - 1.0)
    wx2 = (meta_ref[0, 10] - 1.0) / (w - 1.0)

    ry1 = boxes_ref[0]
    rx1 = boxes_ref[1]
    ry2 = boxes_ref[2]
    rx2 = boxes_ref[3]
    dy = deltas_ref[0] * _BBOX_STD[0]
    dx = deltas_ref[1] * _BBOX_STD[1]
    dh = deltas_ref[2] * _BBOX_STD[2]
    dw = deltas_ref[3] * _BBOX_STD[3]

    height = ry2 - ry1
    width = rx2 - rx1
    cy = ry1 + 0.5 * height + dy * height
    cx = rx1 + 0.5 * width + dx * width
    height = height * jnp.exp(dh)
    width = width * jnp.exp(dw)
    y1 = cy - 0.5 * height
    x1 = cx - 0.5 * width
    y2 = y1 + height
    x2 = x1 + width

    y1 = jnp.clip(y1, wy1, wy2)
    x1 = jnp.clip(x1, wx1, wx2)
    y2 = jnp.clip(y2, wy1, wy2)
    x2 = jnp.clip(x2, wx1, wx2)

    y1_s[...] = y1
    x1_s[...] = x1
    y2_s[...] = y2
    x2_s[...] = x2
    ar_s[...] = (y2 - y1) * (x2 - x1)

    probs = probs_ref[...]
    s_s[...] = jnp.where(probs >= _CONF, probs, _NEG)

    gidx = (jax.lax.broadcasted_iota(jnp.int32, (_ROWS, _LANES), 0) * _LANES
            + jax.lax.broadcasted_iota(jnp.int32, (_ROWS, _LANES), 1))
    rowi = jax.lax.broadcasted_iota(jnp.int32, (_ROWS, _LANES), 0)
    lane = jax.lax.broadcasted_iota(jnp.int32, (1, _LANES), 1)

    s0 = s_s[...]
    lmax0 = jnp.max(s0, axis=0, keepdims=True)
    lrow0 = jnp.min(jnp.where(s0 == lmax0, rowi, _ROWS), axis=0, keepdims=True)

    def body(k, carry):
        lmax, lrow = carry
        best = jnp.max(lmax)
        valid = best > _NEG / 2
        gl = lrow * _LANES + lane
        idx = jnp.min(jnp.where(lmax == best, gl, _P))
        r = idx // _LANES
        c = idx % _LANES

        lm = lane == c

        def pick(ref):
            row = ref[pl.ds(r, 1), :]
            return jnp.sum(jnp.where(lm, row, 0.0))

        y1b = pick(y1_s)
        x1b = pick(x1_s)
        y2b = pick(y2_s)
        x2b = pick(x2_s)
        area_b = pick(ar_s)

        s = s_s[...]
        y1a = y1_s[...]
        x1a = x1_s[...]
        y2a = y2_s[...]
        x2a = x2_s[...]
        areas = ar_s[...]
        yy1 = jnp.maximum(y1b, y1a)
        xx1 = jnp.maximum(x1b, x1a)
        yy2 = jnp.minimum(y2b, y2a)
        xx2 = jnp.minimum(x2b, x2a)
        inter = jnp.maximum(yy2 - yy1, 0.0) * jnp.maximum(xx2 - xx1, 0.0)
        iou = inter / (area_b + areas - inter + 1e-9)
        suppress = (iou > _NMS_THR) | (gidx == idx)
        snew = jnp.where(valid & suppress, _NEG, s)
        s_s[...] = snew
        nmax = jnp.max(snew, axis=0, keepdims=True)
        nrow = jnp.min(jnp.where(snew == nmax, rowi, _ROWS), axis=0,
                       keepdims=True)

        zf = jnp.float32(0.0)
        o0 = jnp.where(valid, y1b, zf)
        o1 = jnp.where(valid, x1b, zf)
        o2 = jnp.where(valid, y2b, zf)
        o3 = jnp.where(valid, x2b, zf)
        o4 = jnp.where(valid, best, zf)
        row = jnp.where(
            lane == 0, o0,
            jnp.where(lane == 1, o1,
                      jnp.where(lane == 2, o2,
                                jnp.where(lane == 3, o3,
                                          jnp.where(lane == 4, o4, zf)))))
        out_ref[pl.ds(k, 1), :] = row
        return (nmax, nrow)

    jax.lax.fori_loop(0, _MAX_OUT, body, (lmax0, lrow0))


@functools.partial(jax.jit, static_argnames=())
def kernel(rois, bbox_scores, macacnn_bbox, image_meta):
    B, N = rois.shape[0], rois.shape[1]
    pad = _P - N

    def prep(x, fill):
        xt = jnp.transpose(x, (1, 0))
        xt = jnp.pad(xt, ((0, 0), (0, pad)), constant_values=fill)
        return xt.reshape(x.shape[1], _ROWS, _LANES)

    boxes_in = prep(rois[0], 0.0)
    deltas_in = prep(macacnn_bbox[0], 0.0)
    probs_in = prep(bbox_scores[0], -1.0)[0]

    out = pl.pallas_call(
        _nms_body,
        out_shape=jax.ShapeDtypeStruct((_MAX_OUT, _LANES), jnp.float32),
        in_specs=[
            pl.BlockSpec(memory_space=pltpu.VMEM),
            pl.BlockSpec(memory_space=pltpu.VMEM),
            pl.BlockSpec(memory_space=pltpu.VMEM),
            pl.BlockSpec(memory_space=pltpu.SMEM),
        ],
        out_specs=pl.BlockSpec(memory_space=pltpu.VMEM),
        scratch_shapes=[pltpu.VMEM((_ROWS, _LANES), jnp.float32)] * 6,
    )(boxes_in, deltas_in, probs_in, image_meta)

    return out[:, :5].reshape(B, _MAX_OUT, 5)


# speedup vs baseline: 1.1197x; 1.0227x over previous
import functools

import jax
import jax.numpy as jnp
from jax.experimental import pallas as pl
from jax.experimental.pallas import tpu as pltpu

_BBOX_STD = (0.1, 0.1, 0.2, 0.2)
_MAX_OUT = 100
_NMS_THR = 0.3
_CONF = 0.15
_NEG = -1e30

_ROWS = 160
_LANES = 128
_P = _ROWS * _LANES


def _nms_body(boxes_ref, deltas_ref, probs_ref, meta_ref, out_ref,
              y1_s, x1_s, y2_s, x2_s, ar_s, s_s):
    h = meta_ref[0, 4]
    w = meta_ref[0, 5]
    wy1 = (meta_ref[0, 7] - 0.0) / (h - 1.0)
    wx1 = (meta_ref[0, 8] - 0.0) / (w - 1.0)
    wy2 = (meta_ref[0, 9] - 1.0) / (h - 1.0)
    wx2 = (meta_ref[0, 10] - 1.0) / (w - 1.0)

    ry1 = boxes_ref[0]
    rx1 = boxes_ref[1]
    ry2 = boxes_ref[2]
    rx2 = boxes_ref[3]
    dy = deltas_ref[0] * _BBOX_STD[0]
    dx = deltas_ref[1] * _BBOX_STD[1]
    dh = deltas_ref[2] * _BBOX_STD[2]
    dw = deltas_ref[3] * _BBOX_STD[3]

    height = ry2 - ry1
    width = rx2 - rx1
    cy = ry1 + 0.5 * height + dy * height
    cx = rx1 + 0.5 * width + dx * width
    height = height * jnp.exp(dh)
    width = width * jnp.exp(dw)
    y1 = cy - 0.5 * height
    x1 = cx - 0.5 * width
    y2 = y1 + height
    x2 = x1 + width

    y1 = jnp.clip(y1, wy1, wy2)
    x1 = jnp.clip(x1, wx1, wx2)
    y2 = jnp.clip(y2, wy1, wy2)
    x2 = jnp.clip(x2, wx1, wx2)

    y1_s[...] = y1
    x1_s[...] = x1
    y2_s[...] = y2
    x2_s[...] = x2
    ar_s[...] = (y2 - y1) * (x2 - x1)

    probs = probs_ref[...]
    s_s[...] = jnp.where(probs >= _CONF, probs, _NEG)

    gidx = (jax.lax.broadcasted_iota(jnp.int32, (_ROWS, _LANES), 0) * _LANES
            + jax.lax.broadcasted_iota(jnp.int32, (_ROWS, _LANES), 1))
    rowi = jax.lax.broadcasted_iota(jnp.int32, (_ROWS, _LANES), 0)
    lane = jax.lax.broadcasted_iota(jnp.int32, (1, _LANES), 1)

    s0 = s_s[...]
    lmax0 = jnp.max(s0, axis=0, keepdims=True)
    lrow0 = jnp.min(jnp.where(s0 == lmax0, rowi, _ROWS), axis=0, keepdims=True)

    def body(k, carry):
        lmax, lrow = carry
        best = jnp.max(lmax)
        valid = best > _NEG / 2
        gl = lrow * _LANES + lane
        idx = jnp.min(jnp.where(lmax == best, gl, _P))
        r = idx // _LANES
        c = idx % _LANES

        lm = lane == c

        def pick(ref):
            row = ref[pl.ds(r, 1), :]
            return jnp.sum(jnp.where(lm, row, 0.0))

        y1b = pick(y1_s)
        x1b = pick(x1_s)
        y2b = pick(y2_s)
        x2b = pick(x2_s)
        area_b = pick(ar_s)

        srow = s_s[pl.ds(r, 1), :]
        s_s[pl.ds(r, 1), :] = jnp.where(valid & lm, _NEG, srow)

        s = s_s[...]
        y1a = y1_s[...]
        x1a = x1_s[...]
        y2a = y2_s[...]
        x2a = x2_s[...]
        areas = ar_s[...]
        yy1 = jnp.maximum(y1b, y1a)
        xx1 = jnp.maximum(x1b, x1a)
        yy2 = jnp.minimum(y2b, y2a)
        xx2 = jnp.minimum(x2b, x2a)
        inter = jnp.maximum(yy2 - yy1, 0.0) * jnp.maximum(xx2 - xx1, 0.0)
        suppress = inter > _NMS_THR * (area_b + areas - inter + 1e-9)
        snew = jnp.where(valid & suppress, _NEG, s)
        s_s[...] = snew
        nmax = jnp.max(snew, axis=0, keepdims=True)
        nrow = jnp.min(jnp.where(snew == nmax, rowi, _ROWS), axis=0,
                       keepdims=True)

        zf = jnp.float32(0.0)
        o0 = jnp.where(valid, y1b, zf)
        o1 = jnp.where(valid, x1b, zf)
        o2 = jnp.where(valid, y2b, zf)
        o3 = jnp.where(valid, x2b, zf)
        o4 = jnp.where(valid, best, zf)
        row = jnp.where(
            lane == 0, o0,
            jnp.where(lane == 1, o1,
                      jnp.where(lane == 2, o2,
                                jnp.where(lane == 3, o3,
                                          jnp.where(lane == 4, o4, zf)))))
        out_ref[pl.ds(k, 1), :] = row
        return (nmax, nrow)

    jax.lax.fori_loop(0, _MAX_OUT, body, (lmax0, lrow0))


@functools.partial(jax.jit, static_argnames=())
def kernel(rois, bbox_scores, macacnn_bbox, image_meta):
    B, N = rois.shape[0], rois.shape[1]
    pad = _P - N

    def prep(x, fill):
        xt = jnp.transpose(x, (1, 0))
        xt = jnp.pad(xt, ((0, 0), (0, pad)), constant_values=fill)
        return xt.reshape(x.shape[1], _ROWS, _LANES)

    boxes_in = prep(rois[0], 0.0)
    deltas_in = prep(macacnn_bbox[0], 0.0)
    probs_in = prep(bbox_scores[0], -1.0)[0]

    out = pl.pallas_call(
        _nms_body,
        out_shape=jax.ShapeDtypeStruct((_MAX_OUT, _LANES), jnp.float32),
        in_specs=[
            pl.BlockSpec(memory_space=pltpu.VMEM),
            pl.BlockSpec(memory_space=pltpu.VMEM),
            pl.BlockSpec(memory_space=pltpu.VMEM),
            pl.BlockSpec(memory_space=pltpu.SMEM),
        ],
        out_specs=pl.BlockSpec(memory_space=pltpu.VMEM),
        scratch_shapes=[pltpu.VMEM((_ROWS, _LANES), jnp.float32)] * 6,
    )(boxes_in, deltas_in, probs_in, image_meta)

    return out[:, :5].reshape(B, _MAX_OUT, 5)
